# baseline (speedup 1.0000x reference)
import jax
import jax.numpy as jnp
from jax import lax
from jax.experimental import pallas as pl
from jax.experimental.pallas import tpu as pltpu
from jax.experimental.pallas import tpu_sc as plsc

N_NODES = 10000
N_EDGES = 320000
DIM = 128
OUT_DIM = 256
NUM_LAYERS = 3

NCORES = 2
NSUB = 16
NTILES = NCORES * NSUB
CHUNK = 128
TPC = 80
NCHUNKS = NTILES * TPC
E_PAD = NCHUNKS * CHUNK
ACC_ROWS = N_NODES + 128
DEG_N = 16384
COPY_ROWS = 624
COPY_REM = N_NODES - COPY_ROWS * NSUB
ZROWS = 48

_f32 = jnp.float32

_MESH = plsc.VectorSubcoreMesh(core_axis_name="c", subcore_axis_name="s",
                               num_cores=NCORES, num_subcores=NSUB)


def _zero_vmem2(ref, rows, cols):
    zeros16 = jnp.zeros((16,), _f32)
    per_row = cols // 16

    def body(i, _):
        r = i // per_row
        col = (i % per_row) * 16
        ref[r, pl.ds(col, 16)] = zeros16
        return 0

    lax.fori_loop(0, rows * per_row, body, 0)


def _agg_body(h_hbm, src_hbm, dst_hbm, part_hbm, acc_sh,
              rows0, rows1, si0, si1, dstb, zbuf,
              g0, g1, i0, i1, s0, s1):
    c = lax.axis_index("c")
    s = lax.axis_index("s")
    w = c * NSUB + s
    base = w * TPC

    pltpu.sync_copy(dst_hbm.at[pl.ds(base, TPC)], dstb)

    _zero_vmem2(zbuf, ZROWS, DIM)
    row0 = s * COPY_ROWS
    for k in range(COPY_ROWS // ZROWS):
        pltpu.sync_copy(zbuf, acc_sh.at[pl.ds(row0 + k * ZROWS, ZROWS)])

    @pl.when(s == NSUB - 1)
    def _():
        for k in range((ACC_ROWS - NSUB * COPY_ROWS) // ZROWS):
            pltpu.sync_copy(
                zbuf, acc_sh.at[pl.ds(NSUB * COPY_ROWS + k * ZROWS, ZROWS)])

    plsc.subcore_barrier()

    def idx_load(k, sv, sem):
        pltpu.async_copy(src_hbm.at[base + k, 0], sv, sem)

    def idx_wait(sv, sem):
        pltpu.make_async_copy(src_hbm.at[0, 0], sv, sem).wait()

    def gather(sv, rows, sem):
        pltpu.async_copy(h_hbm.at[sv], rows, sem)

    def gather_wait(rows, sem):
        pltpu.make_async_copy(h_hbm.at[si0], rows, sem).wait()

    pltpu.sync_copy(src_hbm.at[base, 0], si0)
    idx_load(1, si1, i1)
    gather(si0, rows0, g0)

    def scatter_wait(rows, sem):
        pltpu.make_async_copy(rows, acc_sh.at[dstb.at[0, 0]], sem).wait()

    def stage(k, rows, si, g, i, sc, sin, rowsn, gn, i_n, scn):
        gather_wait(rows, g)

        @pl.when(k + 2 < TPC)
        def _():
            idx_load(k + 2, si, i)

        pltpu.async_copy(rows, acc_sh.at[dstb.at[k, 0]], sc, add=True)

        @pl.when(k + 1 < TPC)
        def _():
            idx_wait(sin, i_n)

            @pl.when(k > 0)
            def _():
                scatter_wait(rowsn, scn)

            gather(sin, rowsn, gn)

    def body(j, _):
        a = 2 * j
        stage(a, rows0, si0, g0, i0, s0, si1, rows1, g1, i1, s1)
        stage(a + 1, rows1, si1, g1, i1, s1, si0, rows0, g0, i0, s0)
        return 0

    lax.fori_loop(0, TPC // 2, body, 0)
    scatter_wait(rows0, s0)
    scatter_wait(rows1, s1)

    plsc.subcore_barrier()

    pltpu.sync_copy(acc_sh.at[pl.ds(row0, COPY_ROWS)],
                    part_hbm.at[c, pl.ds(row0, COPY_ROWS)])

    @pl.when(s == NSUB - 1)
    def _():
        pltpu.sync_copy(acc_sh.at[pl.ds(NSUB * COPY_ROWS, COPY_REM)],
                        part_hbm.at[c, pl.ds(NSUB * COPY_ROWS, COPY_REM)])


_agg = pl.kernel(
    _agg_body,
    out_type=jax.ShapeDtypeStruct((NCORES, N_NODES, DIM), _f32),
    mesh=_MESH,
    scratch_types=[
        pltpu.VMEM_SHARED((ACC_ROWS, DIM), _f32),
        pltpu.VMEM((CHUNK, DIM), _f32),
        pltpu.VMEM((CHUNK, DIM), _f32),
        pltpu.VMEM((CHUNK,), jnp.int32),
        pltpu.VMEM((CHUNK,), jnp.int32),
        pltpu.VMEM((TPC, 1, CHUNK), jnp.int32),
        pltpu.VMEM((ZROWS, DIM), _f32),
    ] + [pltpu.SemaphoreType.DMA] * 6,
)


def _deg_body(dst_hbm, deg_hbm, deg_sh, dstp, ones_v, stage_v, zbuf,
              d0, d1):
    c = lax.axis_index("c")
    s = lax.axis_index("s")
    w = c * NSUB + s
    per = DEG_N // NSUB

    ones16 = jnp.full((16,), 1.0, _f32)
    zeros16 = jnp.zeros((16,), _f32)
    for i in range(CHUNK // 16):
        ones_v[pl.ds(i * 16, 16)] = ones16
        zbuf[pl.ds(i * 16, 16)] = zeros16

    pltpu.sync_copy(dst_hbm.at[pl.ds(w * TPC, TPC)], dstp)
    for k in range(per // CHUNK):
        pltpu.sync_copy(zbuf, deg_sh.at[pl.ds(s * per + k * CHUNK, CHUNK)])

    plsc.subcore_barrier()

    def dscat(k, sem):
        pltpu.async_copy(ones_v, deg_sh.at[dstp.at[k, 0]], sem, add=True)

    def dwait(sem):
        pltpu.make_async_copy(ones_v, deg_sh.at[dstp.at[0, 0]], sem).wait()

    def body(j, _):
        @pl.when(j > 0)
        def _():
            dwait(d0)

        dscat(2 * j, d0)

        @pl.when(j > 0)
        def _():
            dwait(d1)

        dscat(2 * j + 1, d1)
        return 0

    lax.fori_loop(0, TPC // 2, body, 0)
    dwait(d0)
    dwait(d1)

    plsc.subcore_barrier()

    pltpu.sync_copy(deg_sh.at[pl.ds(s * per, per)], stage_v)
    pltpu.sync_copy(stage_v, deg_hbm.at[c, 0, pl.ds(s * per, per)])


_deg = pl.kernel(
    _deg_body,
    out_type=jax.ShapeDtypeStruct((NCORES, 1, DEG_N), _f32),
    mesh=_MESH,
    scratch_types=[
        pltpu.VMEM_SHARED((DEG_N,), _f32),
        pltpu.VMEM((TPC, 1, CHUNK), jnp.int32),
        pltpu.VMEM((CHUNK,), _f32),
        pltpu.VMEM((DEG_N // NSUB,), _f32),
        pltpu.VMEM((CHUNK,), _f32),
        pltpu.SemaphoreType.DMA,
        pltpu.SemaphoreType.DMA,
    ],
)


def _proj_body(x_ref, w_ref, b_ref, o_ref):
    o_ref[...] = (jnp.dot(x_ref[...], w_ref[...],
                          preferred_element_type=_f32) + b_ref[...])


_proj = pl.pallas_call(
    _proj_body,
    grid=(10,),
    in_specs=[
        pl.BlockSpec((N_NODES // 10, DIM), lambda i: (i, 0)),
        pl.BlockSpec((DIM, DIM), lambda i: (0, 0)),
        pl.BlockSpec((1, DIM), lambda i: (0, 0)),
    ],
    out_specs=pl.BlockSpec((N_NODES // 10, DIM), lambda i: (i, 0)),
    out_shape=jax.ShapeDtypeStruct((N_NODES, DIM), _f32),
)


def _invdeg_body(d_ref, o_ref):
    deg = jnp.sum(d_ref[...], axis=0)
    o_ref[...] = (1.0 / jnp.maximum(deg, 1.0))[:, None]


_invdeg = pl.pallas_call(
    _invdeg_body,
    grid=(1,),
    in_specs=[pl.BlockSpec((NCORES, DEG_N), lambda i: (0, 0))],
    out_specs=pl.BlockSpec((DEG_N, 1), lambda i: (0, 0)),
    out_shape=jax.ShapeDtypeStruct((DEG_N, 1), _f32),
)


def _update_body(h_ref, p0_ref, p1_ref, inv_ref, w_ref, b_ref, o_ref):
    msg = (p0_ref[...] + p1_ref[...]) * inv_ref[...]
    o_ref[...] = jnp.maximum(
        h_ref[...] + jnp.dot(msg, w_ref[...], preferred_element_type=_f32)
        + b_ref[...], 0.0)


_update = pl.pallas_call(
    _update_body,
    grid=(10,),
    in_specs=[
        pl.BlockSpec((N_NODES // 10, DIM), lambda i: (i, 0)),
        pl.BlockSpec((N_NODES // 10, DIM), lambda i: (i, 0)),
        pl.BlockSpec((N_NODES // 10, DIM), lambda i: (i, 0)),
        pl.BlockSpec((N_NODES // 10, 1), lambda i: (i, 0)),
        pl.BlockSpec((DIM, DIM), lambda i: (0, 0)),
        pl.BlockSpec((1, DIM), lambda i: (0, 0)),
    ],
    out_specs=pl.BlockSpec((N_NODES // 10, DIM), lambda i: (i, 0)),
    out_shape=jax.ShapeDtypeStruct((N_NODES, DIM), _f32),
)


def _head_body(h_ref, w1_ref, b1_ref, w2_ref, b2_ref, o_ref):
    g = jnp.sum(h_ref[...], axis=0, keepdims=True) * (1.0 / N_NODES)
    hid = jnp.maximum(
        jnp.dot(g, w1_ref[...], preferred_element_type=_f32) + b1_ref[...],
        0.0)
    o_ref[...] = (jnp.dot(hid, w2_ref[...], preferred_element_type=_f32)
                  + b2_ref[...]).reshape(OUT_DIM)


_head = pl.pallas_call(
    _head_body,
    grid=(1,),
    in_specs=[
        pl.BlockSpec((N_NODES, DIM), lambda i: (0, 0)),
        pl.BlockSpec((DIM, DIM), lambda i: (0, 0)),
        pl.BlockSpec((1, DIM), lambda i: (0, 0)),
        pl.BlockSpec((DIM, OUT_DIM), lambda i: (0, 0)),
        pl.BlockSpec((1, OUT_DIM), lambda i: (0, 0)),
    ],
    out_specs=pl.BlockSpec((OUT_DIM,), lambda i: (0,)),
    out_shape=jax.ShapeDtypeStruct((OUT_DIM,), _f32),
)


def kernel(x, edge_index, W_proj, b_proj, W_layers, b_layers, W_p1, b_p1,
           W_p2, b_p2):
    ei = edge_index.astype(jnp.int32)
    lanes = jnp.mod(jnp.arange(E_PAD - N_EDGES, dtype=jnp.int32), 128)
    pad_src = lanes
    pad_dst = N_NODES + lanes
    src = jnp.concatenate([ei[0], pad_src]).reshape(NCHUNKS, 1, CHUNK)
    dst = jnp.concatenate([ei[1], pad_dst]).reshape(NCHUNKS, 1, CHUNK)

    deg2 = _deg(dst)
    invdeg = _invdeg(deg2.reshape(NCORES, DEG_N))[:N_NODES]
    h = _proj(x, W_proj, b_proj.reshape(1, DIM))

    for l in range(NUM_LAYERS):
        part = _agg(h, src, dst)
        h = _update(h, part[0], part[1], invdeg, W_layers[l],
                    b_layers[l].reshape(1, DIM))

    return _head(h, W_p1, b_p1.reshape(1, DIM), W_p2, b_p2.reshape(1, OUT_DIM))


# speedup vs baseline: 1.0388x; 1.0388x over previous
import jax
import jax.numpy as jnp
from jax import lax
from jax.experimental import pallas as pl
from jax.experimental.pallas import tpu as pltpu
from jax.experimental.pallas import tpu_sc as plsc

N_NODES = 10000
N_EDGES = 320000
DIM = 128
OUT_DIM = 256
NUM_LAYERS = 3

NCORES = 2
NSUB = 16
NTILES = NCORES * NSUB
CHUNK = 128
TPC = 80
NCHUNKS = NTILES * TPC
E_PAD = NCHUNKS * CHUNK
ACC_ROWS = N_NODES + 128
DEG_N = 16384
COPY_ROWS = 624
COPY_REM = N_NODES - COPY_ROWS * NSUB
ZROWS = 48

_f32 = jnp.float32

_MESH = plsc.VectorSubcoreMesh(core_axis_name="c", subcore_axis_name="s",
                               num_cores=NCORES, num_subcores=NSUB)


def _zero_vmem2(ref, rows, cols):
    zeros16 = jnp.zeros((16,), _f32)
    per_row = cols // 16

    def body(i, _):
        r = i // per_row
        col = (i % per_row) * 16
        ref[r, pl.ds(col, 16)] = zeros16
        return 0

    lax.fori_loop(0, rows * per_row, body, 0)


def _agg_body(h_hbm, src_hbm, dst_hbm, part_hbm, acc_sh,
              rows0, rows1, si0, si1, dstb, zbuf,
              g0, g1, i0, i1, s0, s1):
    c = lax.axis_index("c")
    s = lax.axis_index("s")
    w = c * NSUB + s
    base = w * TPC

    pltpu.sync_copy(dst_hbm.at[pl.ds(base, TPC)], dstb)

    _zero_vmem2(zbuf, ZROWS, DIM)
    row0 = s * COPY_ROWS
    for k in range(COPY_ROWS // ZROWS):
        pltpu.sync_copy(zbuf, acc_sh.at[pl.ds(row0 + k * ZROWS, ZROWS)])

    @pl.when(s == NSUB - 1)
    def _():
        for k in range((ACC_ROWS - NSUB * COPY_ROWS) // ZROWS):
            pltpu.sync_copy(
                zbuf, acc_sh.at[pl.ds(NSUB * COPY_ROWS + k * ZROWS, ZROWS)])

    plsc.subcore_barrier()

    def idx_load(k, sv, sem):
        pltpu.async_copy(src_hbm.at[base + k, 0], sv, sem)

    def idx_wait(sv, sem):
        pltpu.make_async_copy(src_hbm.at[0, 0], sv, sem).wait()

    def gather(sv, rows, sem):
        pltpu.async_copy(h_hbm.at[sv], rows, sem)

    def gather_wait(rows, sem):
        pltpu.make_async_copy(h_hbm.at[si0], rows, sem).wait()

    pltpu.sync_copy(src_hbm.at[base, 0], si0)
    idx_load(1, si1, i1)
    gather(si0, rows0, g0)

    def scatter_wait(rows, sem):
        pltpu.make_async_copy(rows, acc_sh.at[dstb.at[0, 0]], sem).wait()

    def stage(k, rows, si, g, i, sc, sin, rowsn, gn, i_n, scn):
        gather_wait(rows, g)

        @pl.when(k + 2 < TPC)
        def _():
            idx_load(k + 2, si, i)

        pltpu.async_copy(rows, acc_sh.at[dstb.at[k, 0]], sc, add=True)

        @pl.when(k + 1 < TPC)
        def _():
            idx_wait(sin, i_n)

            @pl.when(k > 0)
            def _():
                scatter_wait(rowsn, scn)

            gather(sin, rowsn, gn)

    def body(j, _):
        a = 2 * j
        stage(a, rows0, si0, g0, i0, s0, si1, rows1, g1, i1, s1)
        stage(a + 1, rows1, si1, g1, i1, s1, si0, rows0, g0, i0, s0)
        return 0

    lax.fori_loop(0, TPC // 2, body, 0)
    scatter_wait(rows0, s0)
    scatter_wait(rows1, s1)

    plsc.subcore_barrier()

    pltpu.sync_copy(acc_sh.at[pl.ds(row0, COPY_ROWS)],
                    part_hbm.at[c, pl.ds(row0, COPY_ROWS)])

    @pl.when(s == NSUB - 1)
    def _():
        pltpu.sync_copy(acc_sh.at[pl.ds(NSUB * COPY_ROWS, COPY_REM)],
                        part_hbm.at[c, pl.ds(NSUB * COPY_ROWS, COPY_REM)])


_agg = pl.kernel(
    _agg_body,
    out_type=jax.ShapeDtypeStruct((NCORES, N_NODES, DIM), _f32),
    mesh=_MESH,
    scratch_types=[
        pltpu.VMEM_SHARED((ACC_ROWS, DIM), _f32),
        pltpu.VMEM((CHUNK, DIM), _f32),
        pltpu.VMEM((CHUNK, DIM), _f32),
        pltpu.VMEM((CHUNK,), jnp.int32),
        pltpu.VMEM((CHUNK,), jnp.int32),
        pltpu.VMEM((TPC, 1, CHUNK), jnp.int32),
        pltpu.VMEM((ZROWS, DIM), _f32),
    ] + [pltpu.SemaphoreType.DMA] * 6,
)


def _deg_body(dst_hbm, deg_hbm, deg_sh, dstp, ones_v, stage_v, zbuf,
              d0, d1):
    c = lax.axis_index("c")
    s = lax.axis_index("s")
    w = c * NSUB + s
    per = DEG_N // NSUB

    ones16 = jnp.full((16,), 1.0, _f32)
    zeros16 = jnp.zeros((16,), _f32)
    for i in range(CHUNK // 16):
        ones_v[pl.ds(i * 16, 16)] = ones16
        zbuf[pl.ds(i * 16, 16)] = zeros16

    pltpu.sync_copy(dst_hbm.at[pl.ds(w * TPC, TPC)], dstp)
    for k in range(per // CHUNK):
        pltpu.sync_copy(zbuf, deg_sh.at[pl.ds(s * per + k * CHUNK, CHUNK)])

    plsc.subcore_barrier()

    def dscat(k, sem):
        pltpu.async_copy(ones_v, deg_sh.at[dstp.at[k, 0]], sem, add=True)

    def dwait(sem):
        pltpu.make_async_copy(ones_v, deg_sh.at[dstp.at[0, 0]], sem).wait()

    def body(j, _):
        @pl.when(j > 0)
        def _():
            dwait(d0)

        dscat(2 * j, d0)

        @pl.when(j > 0)
        def _():
            dwait(d1)

        dscat(2 * j + 1, d1)
        return 0

    lax.fori_loop(0, TPC // 2, body, 0)
    dwait(d0)
    dwait(d1)

    plsc.subcore_barrier()

    pltpu.sync_copy(deg_sh.at[pl.ds(s * per, per)], stage_v)
    pltpu.sync_copy(stage_v, deg_hbm.at[c, 0, pl.ds(s * per, per)])


_deg = pl.kernel(
    _deg_body,
    out_type=jax.ShapeDtypeStruct((NCORES, 1, DEG_N), _f32),
    mesh=_MESH,
    scratch_types=[
        pltpu.VMEM_SHARED((DEG_N,), _f32),
        pltpu.VMEM((TPC, 1, CHUNK), jnp.int32),
        pltpu.VMEM((CHUNK,), _f32),
        pltpu.VMEM((DEG_N // NSUB,), _f32),
        pltpu.VMEM((CHUNK,), _f32),
        pltpu.SemaphoreType.DMA,
        pltpu.SemaphoreType.DMA,
    ],
)


def _proj_body(x_ref, w_ref, b_ref, o_ref):
    o_ref[...] = (jnp.dot(x_ref[...], w_ref[...],
                          preferred_element_type=_f32) + b_ref[...])


_proj = pl.pallas_call(
    _proj_body,
    grid=(10,),
    in_specs=[
        pl.BlockSpec((N_NODES // 10, DIM), lambda i: (i, 0)),
        pl.BlockSpec((DIM, DIM), lambda i: (0, 0)),
        pl.BlockSpec((1, DIM), lambda i: (0, 0)),
    ],
    out_specs=pl.BlockSpec((N_NODES // 10, DIM), lambda i: (i, 0)),
    out_shape=jax.ShapeDtypeStruct((N_NODES, DIM), _f32),
)


def _invdeg_body(d_ref, o_ref):
    deg = jnp.sum(d_ref[...], axis=0)
    o_ref[...] = (1.0 / jnp.maximum(deg, 1.0))[:, None]


_invdeg = pl.pallas_call(
    _invdeg_body,
    grid=(1,),
    in_specs=[pl.BlockSpec((NCORES, DEG_N), lambda i: (0, 0))],
    out_specs=pl.BlockSpec((DEG_N, 1), lambda i: (0, 0)),
    out_shape=jax.ShapeDtypeStruct((DEG_N, 1), _f32),
)


def _update_body(h_ref, p_ref, inv_ref, w_ref, b_ref, o_ref):
    msg = (p_ref[0] + p_ref[1]) * inv_ref[...]
    o_ref[...] = jnp.maximum(
        h_ref[...] + jnp.dot(msg, w_ref[...], preferred_element_type=_f32)
        + b_ref[...], 0.0)


_update = pl.pallas_call(
    _update_body,
    grid=(10,),
    in_specs=[
        pl.BlockSpec((N_NODES // 10, DIM), lambda i: (i, 0)),
        pl.BlockSpec((NCORES, N_NODES // 10, DIM), lambda i: (0, i, 0)),
        pl.BlockSpec((N_NODES // 10, 1), lambda i: (i, 0)),
        pl.BlockSpec((DIM, DIM), lambda i: (0, 0)),
        pl.BlockSpec((1, DIM), lambda i: (0, 0)),
    ],
    out_specs=pl.BlockSpec((N_NODES // 10, DIM), lambda i: (i, 0)),
    out_shape=jax.ShapeDtypeStruct((N_NODES, DIM), _f32),
)


def _head_body(h_ref, w1_ref, b1_ref, w2_ref, b2_ref, o_ref):
    g = jnp.sum(h_ref[...], axis=0, keepdims=True) * (1.0 / N_NODES)
    hid = jnp.maximum(
        jnp.dot(g, w1_ref[...], preferred_element_type=_f32) + b1_ref[...],
        0.0)
    o_ref[...] = (jnp.dot(hid, w2_ref[...], preferred_element_type=_f32)
                  + b2_ref[...]).reshape(OUT_DIM)


_head = pl.pallas_call(
    _head_body,
    grid=(1,),
    in_specs=[
        pl.BlockSpec((N_NODES, DIM), lambda i: (0, 0)),
        pl.BlockSpec((DIM, DIM), lambda i: (0, 0)),
        pl.BlockSpec((1, DIM), lambda i: (0, 0)),
        pl.BlockSpec((DIM, OUT_DIM), lambda i: (0, 0)),
        pl.BlockSpec((1, OUT_DIM), lambda i: (0, 0)),
    ],
    out_specs=pl.BlockSpec((OUT_DIM,), lambda i: (0,)),
    out_shape=jax.ShapeDtypeStruct((OUT_DIM,), _f32),
)


def kernel(x, edge_index, W_proj, b_proj, W_layers, b_layers, W_p1, b_p1,
           W_p2, b_p2):
    ei = edge_index.astype(jnp.int32)
    lanes = jnp.mod(jnp.arange(E_PAD - N_EDGES, dtype=jnp.int32), 128)
    pad_src = lanes
    pad_dst = N_NODES + lanes
    src = jnp.concatenate([ei[0], pad_src]).reshape(NCHUNKS, 1, CHUNK)
    dst = jnp.concatenate([ei[1], pad_dst]).reshape(NCHUNKS, 1, CHUNK)

    deg2 = _deg(dst)
    invdeg = _invdeg(deg2.reshape(NCORES, DEG_N))
    h = _proj(x, W_proj, b_proj.reshape(1, DIM))

    for l in range(NUM_LAYERS):
        part = _agg(h, src, dst)
        h = _update(h, part, invdeg, W_layers[l],
                    b_layers[l].reshape(1, DIM))

    return _head(h, W_p1, b_p1.reshape(1, DIM), W_p2, b_p2.reshape(1, OUT_DIM))
